# baseline (speedup 1.0000x reference)
import jax
import jax.numpy as jnp
from jax.experimental import pallas as pl
from jax.experimental.pallas import tpu as pltpu

N = 512
EMB = 64
HD = 64
D_IN = EMB + 2 * HD
MID = 64
BOT = HD

BI = 8


def _pool_body(cx_ref, cy_ref, nei_ref, lstm_ref, W1_ref, A_ref, bias_ref,
               W2_ref, b2_ref, out_ref, Bj_s, Ci_s):
    k = pl.program_id(0)

    @pl.when(k == 0)
    def _():
        lstm = lstm_ref[...]
        Bj_s[...] = jnp.dot(lstm, W1_ref[EMB:EMB + HD, :],
                            preferred_element_type=jnp.float32) + bias_ref[...]
        Ci_s[...] = jnp.dot(lstm, W1_ref[EMB + HD:, :],
                            preferred_element_type=jnp.float32)

    A0 = A_ref[0:1, :].reshape(1, 1, EMB)
    A1 = A_ref[1:2, :].reshape(1, 1, EMB)
    c = cx_ref[...] * A0 + cy_ref[...] * A1
    h = c + Bj_s[...][None, :, :] + Ci_s[pl.ds(k * BI, BI), :][:, None, :]
    h = jnp.maximum(h, 0.0)
    h2 = h.reshape(BI * N, MID)
    P = jnp.dot(h2, W2_ref[...], preferred_element_type=jnp.float32)
    P = jnp.maximum(P + b2_ref[...], 0.0)
    P3 = P.reshape(BI, N, BOT)
    masked = jnp.where(nei_ref[...] > 0, P3, 0.0)
    out_ref[...] = jnp.max(masked, axis=1)


def kernel(corr_index, nei_index, nei_num, lstm_state, curr_pos_abs,
           W_se, b_se, W1, b1, W2, b2):
    cx = corr_index[:, :, 0:1]
    cy = corr_index[:, :, 1:2]
    nei3 = nei_index[:, :, None]
    A = W_se @ W1[:EMB]
    bias = (b_se @ W1[:EMB] + b1)[None, :]
    b2r = b2[None, :]

    out = pl.pallas_call(
        _pool_body,
        grid=(N // BI,),
        in_specs=[
            pl.BlockSpec((BI, N, 1), lambda k: (k, 0, 0)),
            pl.BlockSpec((BI, N, 1), lambda k: (k, 0, 0)),
            pl.BlockSpec((BI, N, 1), lambda k: (k, 0, 0)),
            pl.BlockSpec((N, HD), lambda k: (0, 0)),
            pl.BlockSpec((D_IN, MID), lambda k: (0, 0)),
            pl.BlockSpec((2, EMB), lambda k: (0, 0)),
            pl.BlockSpec((1, MID), lambda k: (0, 0)),
            pl.BlockSpec((MID, BOT), lambda k: (0, 0)),
            pl.BlockSpec((1, BOT), lambda k: (0, 0)),
        ],
        out_specs=pl.BlockSpec((BI, BOT), lambda k: (k, 0)),
        out_shape=jax.ShapeDtypeStruct((N, BOT), jnp.float32),
        scratch_shapes=[pltpu.VMEM((N, MID), jnp.float32),
                        pltpu.VMEM((N, MID), jnp.float32)],
    )(cx, cy, nei3, lstm_state, W1, A, bias, W2, b2r)
    return out


# speedup vs baseline: 5.4689x; 5.4689x over previous
import jax
import jax.numpy as jnp
from jax.experimental import pallas as pl
from jax.experimental.pallas import tpu as pltpu

N = 512
EMB = 64
HD = 64
D_IN = EMB + 2 * HD
MID = 64
BOT = HD

BI = 8


def _pool_body(cx_ref, cy_ref, nei_ref, lstm_blk_ref, lstmT_ref, W1T_ref,
               At_ref, biasT_ref, W2T_ref, b2c_ref, out_ref, BjT_s):
    k = pl.program_id(0)

    @pl.when(k == 0)
    def _():
        BjT_s[...] = jnp.dot(W1T_ref[:, EMB:EMB + HD], lstmT_ref[...],
                             preferred_element_type=jnp.float32) + biasT_ref[...]

    CiT_blk = jnp.dot(W1T_ref[:, EMB + HD:], lstm_blk_ref[...].T,
                      preferred_element_type=jnp.float32)
    BjT = BjT_s[...]
    W2T = W2T_ref[...]
    b2c = b2c_ref[...]
    A0 = At_ref[:, 0:1]
    A1 = At_ref[:, 1:2]
    cols = []
    for il in range(BI):
        pre = A0 * cx_ref[il:il + 1, :] + A1 * cy_ref[il:il + 1, :]
        h = jnp.maximum(pre + BjT + CiT_blk[:, il:il + 1], 0.0)
        P = jnp.dot(W2T, h, preferred_element_type=jnp.float32)
        P = jnp.maximum(P + b2c, 0.0)
        masked = jnp.where(nei_ref[il:il + 1, :] > 0, P, 0.0)
        cols.append(jnp.max(masked, axis=1, keepdims=True))
    out_ref[...] = jnp.concatenate(cols, axis=1).T


def kernel(corr_index, nei_index, nei_num, lstm_state, curr_pos_abs,
           W_se, b_se, W1, b1, W2, b2):
    cx = corr_index[:, :, 0]
    cy = corr_index[:, :, 1]
    A = W_se @ W1[:EMB]
    At = A.T
    biasT = (b_se @ W1[:EMB] + b1)[:, None]
    W1T = W1.T
    W2T = W2.T
    b2c = b2[:, None]
    lstmT = lstm_state.T

    out = pl.pallas_call(
        _pool_body,
        grid=(N // BI,),
        in_specs=[
            pl.BlockSpec((BI, N), lambda k: (k, 0)),
            pl.BlockSpec((BI, N), lambda k: (k, 0)),
            pl.BlockSpec((BI, N), lambda k: (k, 0)),
            pl.BlockSpec((BI, HD), lambda k: (k, 0)),
            pl.BlockSpec((HD, N), lambda k: (0, 0)),
            pl.BlockSpec((MID, D_IN), lambda k: (0, 0)),
            pl.BlockSpec((MID, 2), lambda k: (0, 0)),
            pl.BlockSpec((MID, 1), lambda k: (0, 0)),
            pl.BlockSpec((BOT, MID), lambda k: (0, 0)),
            pl.BlockSpec((BOT, 1), lambda k: (0, 0)),
        ],
        out_specs=pl.BlockSpec((BI, BOT), lambda k: (k, 0)),
        out_shape=jax.ShapeDtypeStruct((N, BOT), jnp.float32),
        scratch_shapes=[pltpu.VMEM((MID, N), jnp.float32)],
    )(cx, cy, nei_index, lstm_state, lstmT, W1T, At, biasT, W2T, b2c)
    return out


# speedup vs baseline: 6.5075x; 1.1899x over previous
import jax
import jax.numpy as jnp
from jax.experimental import pallas as pl
from jax.experimental.pallas import tpu as pltpu

N = 512
EMB = 64
HD = 64
D_IN = EMB + 2 * HD
MID = 64
BOT = HD

BI = 16


def _pool_body(cx_ref, cy_ref, nei_ref, lstm_blk_ref, lstmT_ref, W1T_ref,
               At_ref, biasT_ref, W2T_ref, b2c_ref, out_ref, BjT_s):
    k = pl.program_id(0)

    @pl.when(k == 0)
    def _():
        BjT_s[...] = jnp.dot(W1T_ref[:, EMB:EMB + HD], lstmT_ref[...],
                             preferred_element_type=jnp.float32) + biasT_ref[...]

    CiT_blk = jnp.dot(W1T_ref[:, EMB + HD:], lstm_blk_ref[...].T,
                      preferred_element_type=jnp.float32)
    BjT = BjT_s[...]
    W2T = W2T_ref[...]
    b2c = b2c_ref[...]
    A0 = At_ref[:, 0:1]
    A1 = At_ref[:, 1:2]
    cols = []
    for il in range(BI):
        pre = A0 * cx_ref[il:il + 1, :] + A1 * cy_ref[il:il + 1, :]
        h = jnp.maximum(pre + BjT + CiT_blk[:, il:il + 1], 0.0)
        P = jnp.dot(W2T, h, preferred_element_type=jnp.float32)
        P = jnp.maximum(P + b2c, 0.0)
        masked = jnp.where(nei_ref[il:il + 1, :] > 0, P, 0.0)
        cols.append(jnp.max(masked, axis=1, keepdims=True))
    out_ref[...] = jnp.concatenate(cols, axis=1).T


def kernel(corr_index, nei_index, nei_num, lstm_state, curr_pos_abs,
           W_se, b_se, W1, b1, W2, b2):
    cx = corr_index[:, :, 0]
    cy = corr_index[:, :, 1]
    A = W_se @ W1[:EMB]
    At = A.T
    biasT = (b_se @ W1[:EMB] + b1)[:, None]
    W1T = W1.T
    W2T = W2.T
    b2c = b2[:, None]
    lstmT = lstm_state.T

    out = pl.pallas_call(
        _pool_body,
        grid=(N // BI,),
        in_specs=[
            pl.BlockSpec((BI, N), lambda k: (k, 0)),
            pl.BlockSpec((BI, N), lambda k: (k, 0)),
            pl.BlockSpec((BI, N), lambda k: (k, 0)),
            pl.BlockSpec((BI, HD), lambda k: (k, 0)),
            pl.BlockSpec((HD, N), lambda k: (0, 0)),
            pl.BlockSpec((MID, D_IN), lambda k: (0, 0)),
            pl.BlockSpec((MID, 2), lambda k: (0, 0)),
            pl.BlockSpec((MID, 1), lambda k: (0, 0)),
            pl.BlockSpec((BOT, MID), lambda k: (0, 0)),
            pl.BlockSpec((BOT, 1), lambda k: (0, 0)),
        ],
        out_specs=pl.BlockSpec((BI, BOT), lambda k: (k, 0)),
        out_shape=jax.ShapeDtypeStruct((N, BOT), jnp.float32),
        scratch_shapes=[pltpu.VMEM((MID, N), jnp.float32)],
    )(cx, cy, nei_index, lstm_state, lstmT, W1T, At, biasT, W2T, b2c)
    return out
